# baseline (speedup 1.0000x reference)
import functools

import jax
import jax.numpy as jnp
from jax import lax
from jax.experimental import pallas as pl
from jax.experimental.pallas import tpu as pltpu
from jax.experimental.pallas import tpu_sc as plsc

_POOL = 2

_NC = 2
_NS = 16
_NW = _NC * _NS


def _make_sc_scatter(B, C, hw, out_h, out_w):
    nplanes = B * C
    planes_per_w = nplanes // _NW
    assert planes_per_w * _NW == nplanes
    assert C % 8 == 0 and out_w % 128 == 0
    groups = hw // 16
    xtiles = out_w // 128

    mesh = plsc.VectorSubcoreMesh(core_axis_name="c", subcore_axis_name="s")

    @functools.partial(
        pl.kernel,
        mesh=mesh,
        out_type=jax.ShapeDtypeStruct(
            (B, out_h, C // 8, xtiles, 8, 128), jnp.float32
        ),
        scratch_types=[
            pltpu.VMEM((hw,), jnp.int32),
            pltpu.VMEM((hw,), jnp.float32),
            pltpu.VMEM((out_h, xtiles, 128), jnp.float32),
            pltpu.SemaphoreType.DMA,
            [pltpu.SemaphoreType.DMA] * 4,
        ],
        compiler_params=pltpu.CompilerParams(needs_layout_passes=False),
    )
    def sc_scatter(mask_hbm, upd_hbm, out_hbm, mvec, uvec, acc, in_sem, out_sems):
        wid = lax.axis_index("s") * _NC + lax.axis_index("c")
        base = wid * planes_per_w
        ych = out_h // 4

        def in_copies(i):
            return (
                pltpu.make_async_copy(mask_hbm.at[base + i], mvec, in_sem),
                pltpu.make_async_copy(upd_hbm.at[base + i], uvec, in_sem),
            )

        def chunk_drain(i, k):
            plane = base + i
            b = plane // C
            c = plane % C
            ct = c // 8
            c8 = c % 8
            return pltpu.make_async_copy(
                acc.at[pl.ds(k * ych, ych)],
                out_hbm.at[b, pl.ds(k * ych, ych), ct, :, c8, :],
                out_sems[k],
            )

        def zero_chunk(k):
            zeros = jnp.zeros((16,), jnp.float32)

            @plsc.parallel_loop(k * ych, (k + 1) * ych, unroll=2)
            def _zbody(y):
                for xt in range(xtiles):
                    for kk in range(8):
                        acc[y, xt, pl.ds(kk * 16, 16)] = zeros

        def scatter_plane():
            @plsc.parallel_loop(0, groups, unroll=16)
            def _sbody(g):
                m = mvec[pl.ds(g * 16, 16)]
                u = uvec[pl.ds(g * 16, 16)]
                q = (m.astype(jnp.uint32) // jnp.uint32(C)).astype(jnp.int32)
                i0 = lax.shift_right_logical(q, 8)
                i1 = lax.bitwise_and(lax.shift_right_logical(q, 7), 1)
                i2 = lax.bitwise_and(q, 127)
                plsc.addupdate_scatter(acc, [i0, i1, i2], u)

        m0, u0 = in_copies(0)
        m0.start()
        u0.start()
        prev_drains = None
        for i in range(planes_per_w):
            if prev_drains is not None:
                for k in range(4):
                    prev_drains[k].wait()
                    zero_chunk(k)
            else:
                for k in range(4):
                    zero_chunk(k)
            mi, ui = in_copies(i)
            mi.wait()
            ui.wait()
            scatter_plane()
            if i + 1 < planes_per_w:
                mn, un = in_copies(i + 1)
                mn.start()
                un.start()
            drains = [chunk_drain(i, k) for k in range(4)]
            for d in drains:
                d.start()
            prev_drains = drains
        for d in prev_drains:
            d.wait()

    return sc_scatter


def kernel(updates, mask):
    B, H, W, C = updates.shape
    hw = H * W
    out_h, out_w = H * _POOL, W * _POOL

    mask = mask.astype(jnp.int32)
    mask_t = jnp.swapaxes(mask.reshape(B, hw, C), 1, 2).reshape(B * C, hw)
    upd_t = jnp.swapaxes(updates.reshape(B, hw, C), 1, 2).reshape(B * C, hw)

    out6 = _make_sc_scatter(B, C, hw, out_h, out_w)(mask_t, upd_t)

    out = out6.transpose(0, 1, 3, 5, 2, 4)
    return out.reshape(B, out_h, out_w, C)


# speedup vs baseline: 1.0432x; 1.0432x over previous
import functools

import jax
import jax.numpy as jnp
from jax import lax
from jax.experimental import pallas as pl
from jax.experimental.pallas import tpu as pltpu
from jax.experimental.pallas import tpu_sc as plsc

_POOL = 2

_NC = 2
_NS = 16
_NW = _NC * _NS


def _make_sc_scatter(B, C, hw, out_h, out_w):
    nplanes = B * C
    planes_per_w = nplanes // _NW
    assert planes_per_w * _NW == nplanes
    assert C % 8 == 0 and out_w % 128 == 0
    groups = hw // 16
    xtiles = out_w // 128

    mesh = plsc.VectorSubcoreMesh(core_axis_name="c", subcore_axis_name="s")

    @functools.partial(
        pl.kernel,
        mesh=mesh,
        out_type=jax.ShapeDtypeStruct(
            (B, out_h, C // 8, xtiles, 8, 128), jnp.float32
        ),
        scratch_types=[
            pltpu.VMEM((hw,), jnp.int32),
            pltpu.VMEM((hw,), jnp.float32),
            pltpu.VMEM((out_h, xtiles, 128), jnp.float32),
            pltpu.SemaphoreType.DMA,
            [pltpu.SemaphoreType.DMA] * 4,
        ],
        compiler_params=pltpu.CompilerParams(needs_layout_passes=False),
    )
    def sc_scatter(mask_hbm, upd_hbm, out_hbm, mvec, uvec, acc, in_sem, out_sems):
        wid = lax.axis_index("s") * _NC + lax.axis_index("c")
        base = wid * planes_per_w
        ych = out_h // 4

        def in_copies(i):
            return (
                pltpu.make_async_copy(mask_hbm.at[base + i], mvec, in_sem),
                pltpu.make_async_copy(upd_hbm.at[base + i], uvec, in_sem),
            )

        def chunk_drain(i, k):
            plane = base + i
            b = plane // C
            c = plane % C
            ct = c // 8
            c8 = c % 8
            return pltpu.make_async_copy(
                acc.at[pl.ds(k * ych, ych)],
                out_hbm.at[b, pl.ds(k * ych, ych), ct, :, c8, :],
                out_sems[k],
            )

        def zero_chunk(k):
            zeros = jnp.zeros((16,), jnp.float32)

            @plsc.parallel_loop(k * ych, (k + 1) * ych, unroll=2)
            def _zbody(y):
                for xt in range(xtiles):
                    for kk in range(8):
                        acc[y, xt, pl.ds(kk * 16, 16)] = zeros

        def scatter_plane():
            @plsc.parallel_loop(0, groups, unroll=8)
            def _sbody(g):
                m = mvec[pl.ds(g * 16, 16)]
                u = uvec[pl.ds(g * 16, 16)]
                q = (m.astype(jnp.uint32) // jnp.uint32(C)).astype(jnp.int32)
                i0 = lax.shift_right_logical(q, 8)
                i1 = lax.bitwise_and(lax.shift_right_logical(q, 7), 1)
                i2 = lax.bitwise_and(q, 127)
                plsc.addupdate_scatter(acc, [i0, i1, i2], u)

        m0, u0 = in_copies(0)
        m0.start()
        u0.start()

        def do_plane(i, carry):
            for k in range(4):
                @pl.when(i > 0)
                def _wait_prev():
                    chunk_drain(i - 1, k).wait()

                zero_chunk(k)
            mi, ui = in_copies(i)
            mi.wait()
            ui.wait()
            scatter_plane()

            @pl.when(i + 1 < planes_per_w)
            def _prefetch_next():
                mn, un = in_copies(i + 1)
                mn.start()
                un.start()

            for k in range(4):
                chunk_drain(i, k).start()
            return carry

        lax.fori_loop(0, planes_per_w, do_plane, 0)
        for k in range(4):
            chunk_drain(planes_per_w - 1, k).wait()

    return sc_scatter


def kernel(updates, mask):
    B, H, W, C = updates.shape
    hw = H * W
    out_h, out_w = H * _POOL, W * _POOL

    mask = mask.astype(jnp.int32)
    mask_t = jnp.swapaxes(mask.reshape(B, hw, C), 1, 2).reshape(B * C, hw)
    upd_t = jnp.swapaxes(updates.reshape(B, hw, C), 1, 2).reshape(B * C, hw)

    out6 = _make_sc_scatter(B, C, hw, out_h, out_w)(mask_t, upd_t)

    out = out6.transpose(0, 1, 3, 5, 2, 4)
    return out.reshape(B, out_h, out_w, C)


# speedup vs baseline: 1.3250x; 1.2702x over previous
import functools

import jax
import jax.numpy as jnp
from jax import lax
from jax.experimental import pallas as pl
from jax.experimental.pallas import tpu as pltpu
from jax.experimental.pallas import tpu_sc as plsc

_POOL = 2

_NC = 2
_NS = 16
_NW = _NC * _NS


def _make_sc_scatter(B, C, H, W, out_h, out_w):
    nplanes = B * C
    planes_per_w = nplanes // _NW
    assert planes_per_w * _NW == nplanes
    assert C % 8 == 0 and out_w % 128 == 0 and W % 16 == 0
    xtiles = out_w // 128
    wgroups = W // 16

    mesh = plsc.VectorSubcoreMesh(core_axis_name="c", subcore_axis_name="s")

    @functools.partial(
        pl.kernel,
        mesh=mesh,
        out_type=jax.ShapeDtypeStruct(
            (B, out_h, C // 8, xtiles, 8, 128), jnp.float32
        ),
        scratch_types=[
            pltpu.VMEM((H, W), jnp.int32),
            pltpu.VMEM((H, W), jnp.float32),
            pltpu.VMEM((out_h, xtiles, 128), jnp.float32),
            pltpu.SemaphoreType.DMA,
            [pltpu.SemaphoreType.DMA] * 4,
        ],
        compiler_params=pltpu.CompilerParams(needs_layout_passes=False),
    )
    def sc_scatter(mask_hbm, upd_hbm, out_hbm, mvec, uvec, acc, in_sem, out_sems):
        wid = lax.axis_index("s") * _NC + lax.axis_index("c")
        base = wid * planes_per_w
        ych = out_h // 4

        def plane_coords(i):
            plane = base + i
            b = plane // C
            c = plane % C
            return b, c // 8, c % 8

        def in_copies(i):
            b, ct, c8 = plane_coords(i)
            return (
                pltpu.make_async_copy(
                    mask_hbm.at[b, :, ct, c8, :], mvec, in_sem
                ),
                pltpu.make_async_copy(
                    upd_hbm.at[b, :, ct, c8, :], uvec, in_sem
                ),
            )

        def chunk_drain(i, k):
            b, ct, c8 = plane_coords(i)
            return pltpu.make_async_copy(
                acc.at[pl.ds(k * ych, ych)],
                out_hbm.at[b, pl.ds(k * ych, ych), ct, :, c8, :],
                out_sems[k],
            )

        def zero_chunk(k):
            zeros = jnp.zeros((16,), jnp.float32)

            @plsc.parallel_loop(k * ych, (k + 1) * ych, unroll=2)
            def _zbody(y):
                for xt in range(xtiles):
                    for kk in range(8):
                        acc[y, xt, pl.ds(kk * 16, 16)] = zeros

        def scatter_plane():
            @plsc.parallel_loop(0, H, unroll=1)
            def _sbody(h):
                for j in range(wgroups):
                    m = mvec[h, pl.ds(j * 16, 16)]
                    u = uvec[h, pl.ds(j * 16, 16)]
                    q = (m.astype(jnp.uint32) // jnp.uint32(C)).astype(
                        jnp.int32
                    )
                    i0 = lax.shift_right_logical(q, 8)
                    i1 = lax.bitwise_and(lax.shift_right_logical(q, 7), 1)
                    i2 = lax.bitwise_and(q, 127)
                    plsc.addupdate_scatter(acc, [i0, i1, i2], u)

        m0, u0 = in_copies(0)
        m0.start()
        u0.start()

        def do_plane(i, carry):
            for k in range(4):
                @pl.when(i > 0)
                def _wait_prev():
                    chunk_drain(i - 1, k).wait()

                zero_chunk(k)
            mi, ui = in_copies(i)
            mi.wait()
            ui.wait()
            scatter_plane()

            @pl.when(i + 1 < planes_per_w)
            def _prefetch_next():
                mn, un = in_copies(i + 1)
                mn.start()
                un.start()

            for k in range(4):
                chunk_drain(i, k).start()
            return carry

        lax.fori_loop(0, planes_per_w, do_plane, 0)
        for k in range(4):
            chunk_drain(planes_per_w - 1, k).wait()

    return sc_scatter


def kernel(updates, mask):
    B, H, W, C = updates.shape
    out_h, out_w = H * _POOL, W * _POOL

    mask = mask.astype(jnp.int32)
    mask5 = jnp.transpose(mask, (0, 1, 3, 2)).reshape(B, H, C // 8, 8, W)
    upd5 = jnp.transpose(updates, (0, 1, 3, 2)).reshape(B, H, C // 8, 8, W)

    out6 = _make_sc_scatter(B, C, H, W, out_h, out_w)(mask5, upd5)

    out = out6.transpose(0, 1, 3, 5, 2, 4)
    return out.reshape(B, out_h, out_w, C)


# speedup vs baseline: 1.3478x; 1.0172x over previous
import functools

import jax
import jax.numpy as jnp
from jax import lax
from jax.experimental import pallas as pl
from jax.experimental.pallas import tpu as pltpu
from jax.experimental.pallas import tpu_sc as plsc

_POOL = 2

_NC = 2
_NS = 16
_NW = _NC * _NS


def _make_sc_scatter(B, C, H, W, out_h, out_w):
    nplanes = B * C
    planes_per_w = nplanes // _NW
    assert planes_per_w * _NW == nplanes
    assert C % 8 == 0 and out_w % 128 == 0 and W % 16 == 0
    xtiles = out_w // 128
    wgroups = W // 16

    mesh = plsc.VectorSubcoreMesh(core_axis_name="c", subcore_axis_name="s")

    @functools.partial(
        pl.kernel,
        mesh=mesh,
        out_type=jax.ShapeDtypeStruct(
            (B, out_h, C // 8, xtiles, 8, 128), jnp.float32
        ),
        scratch_types=[
            pltpu.VMEM((H, W), jnp.int32),
            pltpu.VMEM((H, W), jnp.float32),
            pltpu.VMEM((out_h, xtiles, 128), jnp.float32),
            pltpu.SemaphoreType.DMA,
            [pltpu.SemaphoreType.DMA] * 8,
        ],
        compiler_params=pltpu.CompilerParams(needs_layout_passes=False),
    )
    def sc_scatter(mask_hbm, upd_hbm, out_hbm, mvec, uvec, acc, in_sem, out_sems):
        wid = lax.axis_index("s") * _NC + lax.axis_index("c")
        base = wid * planes_per_w
        ych = out_h // 8

        def plane_coords(i):
            plane = base + i
            b = plane // C
            c = plane % C
            return b, c // 8, c % 8

        def in_copies(i):
            b, ct, c8 = plane_coords(i)
            return (
                pltpu.make_async_copy(
                    mask_hbm.at[b, :, ct, c8, :], mvec, in_sem
                ),
                pltpu.make_async_copy(
                    upd_hbm.at[b, :, ct, c8, :], uvec, in_sem
                ),
            )

        def chunk_drain(i, k):
            b, ct, c8 = plane_coords(i)
            return pltpu.make_async_copy(
                acc.at[pl.ds(k * ych, ych)],
                out_hbm.at[b, pl.ds(k * ych, ych), ct, :, c8, :],
                out_sems[k],
            )

        def zero_chunk(k):
            zeros = jnp.zeros((16,), jnp.float32)

            @plsc.parallel_loop(k * ych, (k + 1) * ych, unroll=2)
            def _zbody(y):
                for xt in range(xtiles):
                    for kk in range(8):
                        acc[y, xt, pl.ds(kk * 16, 16)] = zeros

        def scatter_plane():
            @plsc.parallel_loop(0, H, unroll=1)
            def _sbody(h):
                for j in range(wgroups):
                    m = mvec[h, pl.ds(j * 16, 16)]
                    u = uvec[h, pl.ds(j * 16, 16)]
                    q = (m.astype(jnp.uint32) // jnp.uint32(C)).astype(
                        jnp.int32
                    )
                    i0 = lax.shift_right_logical(q, 8)
                    i1 = lax.bitwise_and(lax.shift_right_logical(q, 7), 1)
                    i2 = lax.bitwise_and(q, 127)
                    plsc.addupdate_scatter(acc, [i0, i1, i2], u)

        m0, u0 = in_copies(0)
        m0.start()
        u0.start()

        def do_plane(i, carry):
            for k in range(8):
                @pl.when(i > 0)
                def _wait_prev():
                    chunk_drain(i - 1, k).wait()

                zero_chunk(k)
            mi, ui = in_copies(i)
            mi.wait()
            ui.wait()
            scatter_plane()

            @pl.when(i + 1 < planes_per_w)
            def _prefetch_next():
                mn, un = in_copies(i + 1)
                mn.start()
                un.start()

            for k in range(8):
                chunk_drain(i, k).start()
            return carry

        lax.fori_loop(0, planes_per_w, do_plane, 0)
        for k in range(8):
            chunk_drain(planes_per_w - 1, k).wait()

    return sc_scatter


def kernel(updates, mask):
    B, H, W, C = updates.shape
    out_h, out_w = H * _POOL, W * _POOL

    mask = mask.astype(jnp.int32)
    mask5 = jnp.transpose(mask, (0, 1, 3, 2)).reshape(B, H, C // 8, 8, W)
    upd5 = jnp.transpose(updates, (0, 1, 3, 2)).reshape(B, H, C // 8, 8, W)

    out6 = _make_sc_scatter(B, C, H, W, out_h, out_w)(mask5, upd5)

    out = out6.transpose(0, 1, 3, 5, 2, 4)
    return out.reshape(B, out_h, out_w, C)


# speedup vs baseline: 1.3479x; 1.0001x over previous
import functools

import jax
import jax.numpy as jnp
from jax import lax
from jax.experimental import pallas as pl
from jax.experimental.pallas import tpu as pltpu
from jax.experimental.pallas import tpu_sc as plsc

_POOL = 2

_NC = 2
_NS = 16
_NW = _NC * _NS


def _make_sc_scatter(B, C, H, W, out_h, out_w):
    nplanes = B * C
    planes_per_w = nplanes // _NW
    assert planes_per_w * _NW == nplanes
    assert C % 8 == 0 and out_w % 128 == 0 and W % 16 == 0
    xtiles = out_w // 128
    wgroups = W // 16

    mesh = plsc.VectorSubcoreMesh(core_axis_name="c", subcore_axis_name="s")

    @functools.partial(
        pl.kernel,
        mesh=mesh,
        out_type=jax.ShapeDtypeStruct(
            (B, out_h, C // 8, xtiles, 8, 128), jnp.float32
        ),
        scratch_types=[
            pltpu.VMEM((H, W), jnp.int32),
            pltpu.VMEM((H, W), jnp.float32),
            pltpu.VMEM((out_h, xtiles, 128), jnp.float32),
            pltpu.SemaphoreType.DMA,
            [pltpu.SemaphoreType.DMA] * 8,
        ],
        compiler_params=pltpu.CompilerParams(needs_layout_passes=False),
    )
    def sc_scatter(mask_hbm, upd_hbm, out_hbm, mvec, uvec, acc, in_sem, out_sems):
        wid = lax.axis_index("s") * _NC + lax.axis_index("c")
        base = wid * planes_per_w
        ych = out_h // 8

        def plane_coords(i):
            plane = base + i
            b = plane // C
            c = plane % C
            return b, c // 8, c % 8

        def in_copies(i):
            b, ct, c8 = plane_coords(i)
            return (
                pltpu.make_async_copy(
                    mask_hbm.at[b, :, ct, c8, :], mvec, in_sem
                ),
                pltpu.make_async_copy(
                    upd_hbm.at[b, :, ct, c8, :], uvec, in_sem
                ),
            )

        def chunk_drain(i, k):
            b, ct, c8 = plane_coords(i)
            return pltpu.make_async_copy(
                acc.at[pl.ds(k * ych, ych)],
                out_hbm.at[b, pl.ds(k * ych, ych), ct, :, c8, :],
                out_sems[k],
            )

        def zero_chunk(k):
            zeros = jnp.zeros((16,), jnp.float32)

            @plsc.parallel_loop(k * ych, (k + 1) * ych, unroll=2)
            def _zbody(y):
                for xt in range(xtiles):
                    for kk in range(8):
                        acc[y, xt, pl.ds(kk * 16, 16)] = zeros

        def scatter_plane():
            @plsc.parallel_loop(0, H, unroll=2)
            def _sbody(h):
                for j in range(wgroups):
                    m = mvec[h, pl.ds(j * 16, 16)]
                    u = uvec[h, pl.ds(j * 16, 16)]
                    q = (m.astype(jnp.uint32) // jnp.uint32(C)).astype(
                        jnp.int32
                    )
                    i0 = lax.shift_right_logical(q, 8)
                    i1 = lax.bitwise_and(lax.shift_right_logical(q, 7), 1)
                    i2 = lax.bitwise_and(q, 127)
                    plsc.addupdate_scatter(acc, [i0, i1, i2], u)

        m0, u0 = in_copies(0)
        m0.start()
        u0.start()

        def do_plane(i, carry):
            for k in range(8):
                @pl.when(i > 0)
                def _wait_prev():
                    chunk_drain(i - 1, k).wait()

                zero_chunk(k)
            mi, ui = in_copies(i)
            mi.wait()
            ui.wait()
            scatter_plane()

            @pl.when(i + 1 < planes_per_w)
            def _prefetch_next():
                mn, un = in_copies(i + 1)
                mn.start()
                un.start()

            for k in range(8):
                chunk_drain(i, k).start()
            return carry

        lax.fori_loop(0, planes_per_w, do_plane, 0)
        for k in range(8):
            chunk_drain(planes_per_w - 1, k).wait()

    return sc_scatter


def kernel(updates, mask):
    B, H, W, C = updates.shape
    out_h, out_w = H * _POOL, W * _POOL

    mask = mask.astype(jnp.int32)
    mask5 = jnp.transpose(mask, (0, 1, 3, 2)).reshape(B, H, C // 8, 8, W)
    upd5 = jnp.transpose(updates, (0, 1, 3, 2)).reshape(B, H, C // 8, 8, W)

    out6 = _make_sc_scatter(B, C, H, W, out_h, out_w)(mask5, upd5)

    out = out6.transpose(0, 1, 3, 5, 2, 4)
    return out.reshape(B, out_h, out_w, C)
